# baseline (speedup 1.0000x reference)
import dataclasses
import functools

import jax
import jax.numpy as jnp
from jax import lax
from jax.experimental import pallas as pl
from jax.experimental.pallas import tpu as pltpu
from jax.experimental.pallas import tpu_sc as plsc

N = 10000
E = 320000
F_IN = 128
HIDDEN = 64
C = 40

NC = 2
NS = 16
LANES = 16
NW = NC * NS

CH = 128
CHN = 80
EPW = CH * CHN
E_PAD = EPW * NW
NP = 10240
ROWS_PER_SUB = NP // NS
D = 128

_MESH = plsc.VectorSubcoreMesh(core_axis_name="c", subcore_axis_name="s")

_CP = pltpu.CompilerParams()
if "needs_layout_passes" in pltpu.CompilerParams.__dataclass_fields__:
    _CP = dataclasses.replace(_CP, needs_layout_passes=False)


@functools.partial(
    pl.kernel,
    out_type=jax.ShapeDtypeStruct((NW, NP), jnp.float32),
    mesh=_MESH,
    scratch_types=[
        pltpu.VMEM((CHN, 2, CH), jnp.int32),
        pltpu.VMEM((NP,), jnp.float32),
    ],
    compiler_params=_CP,
)
def _deg_pass(e_hbm, out_hbm, e_v, deg_v):
    c = lax.axis_index("c")
    s = lax.axis_index("s")
    wid = c * NS + s

    @pl.loop(0, NP // LANES)
    def _(i):
        deg_v[pl.ds(i * LANES, LANES)] = jnp.zeros((LANES,), jnp.float32)

    pltpu.sync_copy(e_hbm.at[pl.ds(wid * CHN, CHN)], e_v)

    @pl.loop(0, CHN)
    def _(j):
        @pl.loop(0, CH // LANES)
        def _(k):
            idx = e_v[j, 1, pl.ds(k * LANES, LANES)]
            cnt, last = plsc.scan_count(idx)
            plsc.addupdate_scatter(deg_v, [idx], cnt.astype(jnp.float32),
                                   mask=last)

    pltpu.sync_copy(deg_v, out_hbm.at[wid])


BI = 8
NB = CHN // BI
TOT_CHUNKS = E_PAD // CH
CPW = TOT_CHUNKS // NS
NBW = CPW // BI


@functools.partial(
    pl.kernel,
    out_type=jax.ShapeDtypeStruct((NP, D), jnp.float32),
    mesh=_MESH,
    scratch_types=[
        pltpu.VMEM((BI, 2, CH), jnp.int32),
        pltpu.VMEM((CH, D), jnp.float32),
        pltpu.VMEM((CH, D), jnp.float32),
        pltpu.VMEM_SHARED((NP, D), jnp.float32),
        pltpu.SemaphoreType.DMA,
        pltpu.SemaphoreType.DMA,
        pltpu.SemaphoreType.DMA,
        pltpu.SemaphoreType.DMA,
    ],
)
def _agg_pass(t_hbm, e_hbm, out_hbm,
              e_v, rows_a, rows_b, agg_sh, gsem_a, gsem_b, ssem_a, ssem_b):
    c = lax.axis_index("c")
    s = lax.axis_index("s")

    @pl.when(c == 0)
    def _():
        @pl.loop(0, CH)
        def _(i):
            @pl.loop(0, D // LANES)
            def _(k):
                rows_a[i, pl.ds(k * LANES, LANES)] = jnp.zeros((LANES,),
                                                               jnp.float32)

        @pl.loop(0, ROWS_PER_SUB // CH)
        def _(r):
            pltpu.sync_copy(rows_a,
                            agg_sh.at[pl.ds(s * ROWS_PER_SUB + r * CH, CH)])

        plsc.subcore_barrier()

        @pl.loop(0, NBW)
        def _(g):
            pltpu.sync_copy(e_hbm.at[pl.ds(s * CPW + g * BI, BI)], e_v)
            bufs = ((rows_a, gsem_a, ssem_a), (rows_b, gsem_b, ssem_b))

            def gather(j, buf, gsem):
                pltpu.async_copy(t_hbm.at[e_v.at[j, 0]], buf, gsem)

            def wait_gather(j, buf, gsem):
                pltpu.make_async_copy(t_hbm.at[e_v.at[j, 0]], buf, gsem).wait()

            def scatter(j, buf, ssem):
                pltpu.async_copy(buf, agg_sh.at[e_v.at[j, 1]], ssem, add=True)

            def wait_scatter(j, buf, ssem):
                pltpu.make_async_copy(buf, agg_sh.at[e_v.at[j, 1]],
                                      ssem).wait()

            gather(0, *bufs[0][:2])
            gather(1, *bufs[1][:2])
            for j in range(BI):
                buf, gsem, ssem = bufs[j % 2]
                wait_gather(j, buf, gsem)
                scatter(j, buf, ssem)
                if j + 2 < BI:
                    wait_scatter(j, buf, ssem)
                    gather(j + 2, buf, gsem)
            wait_scatter(BI - 2, rows_a, ssem_a)
            wait_scatter(BI - 1, rows_b, ssem_b)

        plsc.subcore_barrier()

        @pl.loop(0, ROWS_PER_SUB // CH)
        def _(r):
            off = s * ROWS_PER_SUB + r * CH
            pltpu.sync_copy(agg_sh.at[pl.ds(off, CH)],
                            out_hbm.at[pl.ds(off, CH)])


BLK = 1024
GRID = NP // BLK


def _tc1_body(x_ref, w1_ref, deg_ref, t1_ref, dinv_ref):
    deg = jnp.sum(deg_ref[...], axis=0)[:, None] + 1.0
    dinv = lax.rsqrt(deg)
    h = jnp.dot(x_ref[...], w1_ref[...], preferred_element_type=jnp.float32)
    t1_ref[...] = h * dinv
    dinv_ref[...] = dinv


def _tc2_body(agg_ref, t1_ref, dinv_ref, b1_ref, w2_ref, t2_ref):
    out1 = ((agg_ref[...] + t1_ref[...]) * dinv_ref[...]
            + b1_ref[...])
    r = jnp.maximum(out1, 0.0)
    h2 = jnp.dot(r, w2_ref[...], preferred_element_type=jnp.float32)
    t2_ref[...] = h2 * dinv_ref[...]


def _tc3_body(agg_ref, t2_ref, dinv_ref, b2_ref, out_ref):
    z = (agg_ref[...] + t2_ref[...]) * dinv_ref[...]
    z40 = z[:, :C] + b2_ref[...]
    m = jnp.max(z40, axis=1, keepdims=True)
    lse = jnp.log(jnp.sum(jnp.exp(z40 - m), axis=1, keepdims=True))
    out_ref[...] = z40 - m - lse


def kernel(x, edge_index, W1, b1, W2, b2):
    src = edge_index[0]
    dst = edge_index[1]
    pad = E_PAD - E
    fill = jnp.full((pad,), N, jnp.int32)
    srcp = jnp.concatenate([src, fill]).reshape(TOT_CHUNKS, CH)
    dstp = jnp.concatenate([dst, fill]).reshape(TOT_CHUNKS, CH)
    ep = jnp.stack([srcp, dstp], axis=1)
    xp = jnp.pad(x, ((0, NP - N), (0, 0)))
    w1p = jnp.pad(W1, ((0, 0), (0, D - HIDDEN)))
    b1p = jnp.pad(b1, (0, D - HIDDEN)).reshape(1, D)
    w2p = jnp.pad(W2, ((0, HIDDEN), (0, D - C)))

    degs = _deg_pass(ep)

    t1, dinv = pl.pallas_call(
        _tc1_body,
        grid=(GRID,),
        in_specs=[
            pl.BlockSpec((BLK, F_IN), lambda i: (i, 0)),
            pl.BlockSpec((F_IN, D), lambda i: (0, 0)),
            pl.BlockSpec((NW, BLK), lambda i: (0, i)),
        ],
        out_specs=[
            pl.BlockSpec((BLK, D), lambda i: (i, 0)),
            pl.BlockSpec((BLK, 1), lambda i: (i, 0)),
        ],
        out_shape=[
            jax.ShapeDtypeStruct((NP, D), jnp.float32),
            jax.ShapeDtypeStruct((NP, 1), jnp.float32),
        ],
    )(xp, w1p, degs)

    agg1 = _agg_pass(t1, ep)

    t2 = pl.pallas_call(
        _tc2_body,
        grid=(GRID,),
        in_specs=[
            pl.BlockSpec((BLK, D), lambda i: (i, 0)),
            pl.BlockSpec((BLK, D), lambda i: (i, 0)),
            pl.BlockSpec((BLK, 1), lambda i: (i, 0)),
            pl.BlockSpec((1, D), lambda i: (0, 0)),
            pl.BlockSpec((D, D), lambda i: (0, 0)),
        ],
        out_specs=pl.BlockSpec((BLK, D), lambda i: (i, 0)),
        out_shape=jax.ShapeDtypeStruct((NP, D), jnp.float32),
    )(agg1, t1, dinv, b1p, w2p)

    agg2 = _agg_pass(t2, ep)

    outp = pl.pallas_call(
        _tc3_body,
        grid=(GRID,),
        in_specs=[
            pl.BlockSpec((BLK, D), lambda i: (i, 0)),
            pl.BlockSpec((BLK, D), lambda i: (i, 0)),
            pl.BlockSpec((BLK, 1), lambda i: (i, 0)),
            pl.BlockSpec((1, C), lambda i: (0, 0)),
        ],
        out_specs=pl.BlockSpec((BLK, C), lambda i: (i, 0)),
        out_shape=jax.ShapeDtypeStruct((NP, C), jnp.float32),
    )(agg2, t2, dinv, b2.reshape(1, C))

    return outp[:N]


# speedup vs baseline: 2.0109x; 2.0109x over previous
import dataclasses
import functools

import jax
import jax.numpy as jnp
from jax import lax
from jax.experimental import pallas as pl
from jax.experimental.pallas import tpu as pltpu
from jax.experimental.pallas import tpu_sc as plsc

N = 10000
E = 320000
F_IN = 128
HIDDEN = 64
C = 40

NC = 2
NS = 16
LANES = 16
NW = NC * NS

CH = 128
CHN = 80
EPW = CH * CHN
E_PAD = EPW * NW
NP = 10240
ROWS_PER_SUB = NP // NS
DA = 64

BI = 8
TOT_CHUNKS = E_PAD // CH
NB0, NB1 = 16, 4
CPW0, CPW1 = NB0 * BI, NB1 * BI
assert NS * (CPW0 + CPW1) == TOT_CHUNKS

_MESH = plsc.VectorSubcoreMesh(core_axis_name="c", subcore_axis_name="s")

_CP = pltpu.CompilerParams()
if "needs_layout_passes" in pltpu.CompilerParams.__dataclass_fields__:
    _CP = dataclasses.replace(_CP, needs_layout_passes=False)
_CP_AGG = dataclasses.replace(pltpu.CompilerParams(),
                              use_tc_tiling_on_sc=False)


@functools.partial(
    pl.kernel,
    out_type=jax.ShapeDtypeStruct((NW, NP), jnp.float32),
    mesh=_MESH,
    scratch_types=[
        pltpu.VMEM((CHN, 2, CH), jnp.int32),
        pltpu.VMEM((NP,), jnp.float32),
    ],
    compiler_params=_CP,
)
def _deg_pass(e_hbm, out_hbm, e_v, deg_v):
    c = lax.axis_index("c")
    s = lax.axis_index("s")
    wid = c * NS + s

    @pl.loop(0, NP // LANES)
    def _(i):
        deg_v[pl.ds(i * LANES, LANES)] = jnp.zeros((LANES,), jnp.float32)

    pltpu.sync_copy(e_hbm.at[pl.ds(wid * CHN, CHN)], e_v)

    @pl.loop(0, CHN)
    def _(j):
        @pl.loop(0, CH // LANES)
        def _(k):
            idx = e_v[j, 1, pl.ds(k * LANES, LANES)]
            cnt, last = plsc.scan_count(idx)
            plsc.addupdate_scatter(deg_v, [idx], cnt.astype(jnp.float32),
                                   mask=last)

    pltpu.sync_copy(deg_v, out_hbm.at[wid])


@functools.partial(
    pl.kernel,
    out_type=jax.ShapeDtypeStruct((NC, NP, DA), jnp.float32),
    mesh=_MESH,
    scratch_types=[
        pltpu.VMEM((BI, 2, CH), jnp.int32),
        pltpu.VMEM((CH, DA), jnp.float32),
        pltpu.VMEM((CH, DA), jnp.float32),
        pltpu.VMEM_SHARED((NP, DA), jnp.float32),
        pltpu.SemaphoreType.DMA,
        pltpu.SemaphoreType.DMA,
        pltpu.SemaphoreType.DMA,
        pltpu.SemaphoreType.DMA,
    ],
    compiler_params=_CP_AGG,
)
def _agg_pass(t_hbm, e_hbm, out_hbm,
              e_v, rows_a, rows_b, agg_sh, gsem_a, gsem_b, ssem_a, ssem_b):
    c = lax.axis_index("c")
    s = lax.axis_index("s")

    @pl.loop(0, CH)
    def _(i):
        @pl.loop(0, DA // LANES)
        def _(k):
            rows_a[i, pl.ds(k * LANES, LANES)] = jnp.zeros((LANES,),
                                                           jnp.float32)

    @pl.loop(0, ROWS_PER_SUB // CH)
    def _(r):
        pltpu.sync_copy(rows_a,
                        agg_sh.at[pl.ds(s * ROWS_PER_SUB + r * CH, CH)])

    plsc.subcore_barrier()

    start_chunk = jnp.where(c == 0, s * CPW0, NS * CPW0 + s * CPW1)
    nblocks = jnp.where(c == 0, NB0, NB1)

    @pl.loop(0, nblocks)
    def _(g):
        pltpu.sync_copy(e_hbm.at[pl.ds(start_chunk + g * BI, BI)], e_v)
        bufs = ((rows_a, gsem_a, ssem_a), (rows_b, gsem_b, ssem_b))

        def gather(j, buf, gsem):
            pltpu.async_copy(t_hbm.at[e_v.at[j, 0]], buf, gsem)

        def wait_gather(j, buf, gsem):
            pltpu.make_async_copy(t_hbm.at[e_v.at[j, 0]], buf, gsem).wait()

        def scatter(j, buf, ssem):
            pltpu.async_copy(buf, agg_sh.at[e_v.at[j, 1]], ssem, add=True)

        def wait_scatter(j, buf, ssem):
            pltpu.make_async_copy(buf, agg_sh.at[e_v.at[j, 1]], ssem).wait()

        gather(0, *bufs[0][:2])
        gather(1, *bufs[1][:2])
        for j in range(BI):
            buf, gsem, ssem = bufs[j % 2]
            wait_gather(j, buf, gsem)
            scatter(j, buf, ssem)
            if j + 2 < BI:
                wait_scatter(j, buf, ssem)
                gather(j + 2, buf, gsem)
        wait_scatter(BI - 2, rows_a, ssem_a)
        wait_scatter(BI - 1, rows_b, ssem_b)

    plsc.subcore_barrier()

    @pl.loop(0, ROWS_PER_SUB // CH)
    def _(r):
        off = s * ROWS_PER_SUB + r * CH
        pltpu.sync_copy(agg_sh.at[pl.ds(off, CH)],
                        out_hbm.at[c].at[pl.ds(off, CH)])


BLK = 1024
GRID = NP // BLK


def _tc1_body(x_ref, w1_ref, deg_ref, t1_ref, dinv_ref):
    deg = jnp.sum(deg_ref[...], axis=0)[:, None] + 1.0
    dinv = lax.rsqrt(deg)
    h = jnp.dot(x_ref[...], w1_ref[...], preferred_element_type=jnp.float32)
    t1_ref[...] = h * dinv
    dinv_ref[...] = dinv


def _tc2_body(aga_ref, agb_ref, t1_ref, dinv_ref, b1_ref, w2_ref, t2_ref):
    out1 = ((aga_ref[...] + agb_ref[...] + t1_ref[...]) * dinv_ref[...]
            + b1_ref[...])
    r = jnp.maximum(out1, 0.0)
    h2 = jnp.dot(r, w2_ref[...], preferred_element_type=jnp.float32)
    t2_ref[...] = h2 * dinv_ref[...]


def _tc3_body(aga_ref, agb_ref, t2_ref, dinv_ref, b2_ref, out_ref):
    z = (aga_ref[...] + agb_ref[...] + t2_ref[...]) * dinv_ref[...]
    z40 = z[:, :C] + b2_ref[...]
    m = jnp.max(z40, axis=1, keepdims=True)
    lse = jnp.log(jnp.sum(jnp.exp(z40 - m), axis=1, keepdims=True))
    out_ref[...] = z40 - m - lse


def kernel(x, edge_index, W1, b1, W2, b2):
    src = edge_index[0]
    dst = edge_index[1]
    pad = E_PAD - E
    fill = jnp.full((pad,), N, jnp.int32)
    srcp = jnp.concatenate([src, fill]).reshape(TOT_CHUNKS, CH)
    dstp = jnp.concatenate([dst, fill]).reshape(TOT_CHUNKS, CH)
    ep = jnp.stack([srcp, dstp], axis=1)
    xp = jnp.pad(x, ((0, NP - N), (0, 0)))
    b1p = b1.reshape(1, HIDDEN)
    w2p = jnp.pad(W2, ((0, 0), (0, DA - C)))

    degs = _deg_pass(ep)

    t1, dinv = pl.pallas_call(
        _tc1_body,
        grid=(GRID,),
        in_specs=[
            pl.BlockSpec((BLK, F_IN), lambda i: (i, 0)),
            pl.BlockSpec((F_IN, HIDDEN), lambda i: (0, 0)),
            pl.BlockSpec((NW, BLK), lambda i: (0, i)),
        ],
        out_specs=[
            pl.BlockSpec((BLK, DA), lambda i: (i, 0)),
            pl.BlockSpec((BLK, 1), lambda i: (i, 0)),
        ],
        out_shape=[
            jax.ShapeDtypeStruct((NP, DA), jnp.float32),
            jax.ShapeDtypeStruct((NP, 1), jnp.float32),
        ],
    )(xp, W1, degs)

    agg1 = _agg_pass(t1, ep)

    t2 = pl.pallas_call(
        _tc2_body,
        grid=(GRID,),
        in_specs=[
            pl.BlockSpec((BLK, DA), lambda i: (i, 0)),
            pl.BlockSpec((BLK, DA), lambda i: (i, 0)),
            pl.BlockSpec((BLK, DA), lambda i: (i, 0)),
            pl.BlockSpec((BLK, 1), lambda i: (i, 0)),
            pl.BlockSpec((1, HIDDEN), lambda i: (0, 0)),
            pl.BlockSpec((HIDDEN, DA), lambda i: (0, 0)),
        ],
        out_specs=pl.BlockSpec((BLK, DA), lambda i: (i, 0)),
        out_shape=jax.ShapeDtypeStruct((NP, DA), jnp.float32),
    )(agg1[0], agg1[1], t1, dinv, b1p, w2p)

    agg2 = _agg_pass(t2, ep)

    outp = pl.pallas_call(
        _tc3_body,
        grid=(GRID,),
        in_specs=[
            pl.BlockSpec((BLK, DA), lambda i: (i, 0)),
            pl.BlockSpec((BLK, DA), lambda i: (i, 0)),
            pl.BlockSpec((BLK, DA), lambda i: (i, 0)),
            pl.BlockSpec((BLK, 1), lambda i: (i, 0)),
            pl.BlockSpec((1, C), lambda i: (0, 0)),
        ],
        out_specs=pl.BlockSpec((BLK, C), lambda i: (i, 0)),
        out_shape=jax.ShapeDtypeStruct((NP, C), jnp.float32),
    )(agg2[0], agg2[1], t2, dinv, b2.reshape(1, C))

    return outp[:N]


# speedup vs baseline: 3.0979x; 1.5405x over previous
import dataclasses
import functools

import jax
import jax.numpy as jnp
from jax import lax
from jax.experimental import pallas as pl
from jax.experimental.pallas import tpu as pltpu
from jax.experimental.pallas import tpu_sc as plsc

N = 10000
E = 320000
F_IN = 128
HIDDEN = 64
C = 40

NC = 2
NS = 16
LANES = 16
NW = NC * NS

CH = 128
CHN = 80
EPW = CH * CHN
E_PAD = EPW * NW
NP = 10240
ROWS_PER_SUB = NP // NS
DA = 64

BI = 8
TOT_CHUNKS = E_PAD // CH
NB0, NB1 = 16, 4
CPW0, CPW1 = NB0 * BI, NB1 * BI
assert NS * (CPW0 + CPW1) == TOT_CHUNKS

_MESH = plsc.VectorSubcoreMesh(core_axis_name="c", subcore_axis_name="s")

_CP = pltpu.CompilerParams()
if "needs_layout_passes" in pltpu.CompilerParams.__dataclass_fields__:
    _CP = dataclasses.replace(_CP, needs_layout_passes=False)
_CP_AGG = dataclasses.replace(pltpu.CompilerParams(),
                              use_tc_tiling_on_sc=False)


@functools.partial(
    pl.kernel,
    out_type=jax.ShapeDtypeStruct((NW, NP), jnp.float32),
    mesh=_MESH,
    scratch_types=[
        pltpu.VMEM((CHN, CH), jnp.int32),
        pltpu.VMEM((NP,), jnp.float32),
    ],
    compiler_params=_CP,
)
def _deg_pass(e_hbm, out_hbm, e_v, deg_v):
    c = lax.axis_index("c")
    s = lax.axis_index("s")
    wid = c * NS + s

    @pl.loop(0, NP // LANES)
    def _(i):
        deg_v[pl.ds(i * LANES, LANES)] = jnp.zeros((LANES,), jnp.float32)

    pltpu.sync_copy(e_hbm.at[1].at[pl.ds(wid * CHN, CHN)], e_v)

    @pl.loop(0, CHN)
    def _(j):
        @pl.loop(0, CH // LANES)
        def _(k):
            idx = e_v[j, pl.ds(k * LANES, LANES)]
            cnt, last = plsc.scan_count(idx)
            plsc.addupdate_scatter(deg_v, [idx], cnt.astype(jnp.float32),
                                   mask=last)

    pltpu.sync_copy(deg_v, out_hbm.at[wid])


@functools.partial(
    pl.kernel,
    out_type=jax.ShapeDtypeStruct((NC, NP, DA), jnp.float32),
    mesh=_MESH,
    scratch_types=[
        pltpu.VMEM((BI, CH), jnp.int32),
        pltpu.VMEM((BI, CH), jnp.int32),
        pltpu.VMEM((CH, DA), jnp.float32),
        pltpu.VMEM((CH, DA), jnp.float32),
        pltpu.VMEM_SHARED((NP, DA), jnp.float32),
        pltpu.VMEM_SHARED((NP, DA), jnp.float32),
        pltpu.SemaphoreType.DMA,
        pltpu.SemaphoreType.DMA,
        pltpu.SemaphoreType.DMA,
        pltpu.SemaphoreType.DMA,
    ],
    compiler_params=_CP_AGG,
)
def _agg_pass(t_hbm, e_hbm, out_hbm, src_v, dst_v, rows_a, rows_b,
              agg_sh, t_sh, gsem_a, gsem_b, ssem_a, ssem_b):
    c = lax.axis_index("c")
    s = lax.axis_index("s")

    @pl.loop(0, CH)
    def _(i):
        @pl.loop(0, DA // LANES)
        def _(k):
            rows_a[i, pl.ds(k * LANES, LANES)] = jnp.zeros((LANES,),
                                                           jnp.float32)

    @pl.loop(0, ROWS_PER_SUB // CH)
    def _(r):
        pltpu.sync_copy(rows_a,
                        agg_sh.at[pl.ds(s * ROWS_PER_SUB + r * CH, CH)])

    @pl.when(c == 1)
    def _():
        pltpu.sync_copy(t_hbm.at[pl.ds(s * ROWS_PER_SUB, ROWS_PER_SUB)],
                        t_sh.at[pl.ds(s * ROWS_PER_SUB, ROWS_PER_SUB)])

    plsc.subcore_barrier()

    def edge_loop(t_src, start_chunk, nblocks):
        @pl.loop(0, nblocks)
        def _(g):
            base = start_chunk + g * BI
            pltpu.sync_copy(e_hbm.at[0].at[pl.ds(base, BI)], src_v)
            pltpu.sync_copy(e_hbm.at[1].at[pl.ds(base, BI)], dst_v)
            bufs = ((rows_a, gsem_a, ssem_a), (rows_b, gsem_b, ssem_b))

            def gather(j, buf, gsem):
                pltpu.async_copy(t_src.at[src_v.at[j]], buf, gsem)

            def wait_gather(j, buf, gsem):
                pltpu.make_async_copy(t_src.at[src_v.at[j]], buf, gsem).wait()

            def scatter(j, buf, ssem):
                pltpu.async_copy(buf, agg_sh.at[dst_v.at[j]], ssem, add=True)

            def wait_scatter(j, buf, ssem):
                pltpu.make_async_copy(buf, agg_sh.at[dst_v.at[j]],
                                      ssem).wait()

            gather(0, *bufs[0][:2])
            gather(1, *bufs[1][:2])
            for j in range(BI):
                buf, gsem, ssem = bufs[j % 2]
                wait_gather(j, buf, gsem)
                scatter(j, buf, ssem)
                if j + 2 < BI:
                    wait_scatter(j, buf, ssem)
                    gather(j + 2, buf, gsem)
            wait_scatter(BI - 2, rows_a, ssem_a)
            wait_scatter(BI - 1, rows_b, ssem_b)

    @pl.when(c == 0)
    def _():
        edge_loop(t_hbm, s * CPW0, NB0)

    @pl.when(c == 1)
    def _():
        edge_loop(t_sh, NS * CPW0 + s * CPW1, NB1)

    plsc.subcore_barrier()

    @pl.loop(0, ROWS_PER_SUB // CH)
    def _(r):
        off = s * ROWS_PER_SUB + r * CH
        pltpu.sync_copy(agg_sh.at[pl.ds(off, CH)],
                        out_hbm.at[c].at[pl.ds(off, CH)])


BLK = 1024
GRID = NP // BLK


def _tc1_body(x_ref, w1_ref, deg_ref, t1_ref, dinv_ref):
    deg = jnp.sum(deg_ref[...], axis=0)[:, None] + 1.0
    dinv = lax.rsqrt(deg)
    h = jnp.dot(x_ref[...], w1_ref[...], preferred_element_type=jnp.float32)
    t1_ref[...] = h * dinv
    dinv_ref[...] = dinv


def _tc2_body(aga_ref, agb_ref, t1_ref, dinv_ref, b1_ref, w2_ref, t2_ref):
    out1 = ((aga_ref[...] + agb_ref[...] + t1_ref[...]) * dinv_ref[...]
            + b1_ref[...])
    r = jnp.maximum(out1, 0.0)
    h2 = jnp.dot(r, w2_ref[...], preferred_element_type=jnp.float32)
    t2_ref[...] = h2 * dinv_ref[...]


def _tc3_body(aga_ref, agb_ref, t2_ref, dinv_ref, b2_ref, out_ref):
    z = (aga_ref[...] + agb_ref[...] + t2_ref[...]) * dinv_ref[...]
    z40 = z[:, :C] + b2_ref[...]
    m = jnp.max(z40, axis=1, keepdims=True)
    lse = jnp.log(jnp.sum(jnp.exp(z40 - m), axis=1, keepdims=True))
    out_ref[...] = z40 - m - lse


def kernel(x, edge_index, W1, b1, W2, b2):
    pad = E_PAD - E
    fill = jnp.full((2, pad), N, jnp.int32)
    ep = jnp.concatenate([edge_index, fill], axis=1).reshape(
        2, TOT_CHUNKS, CH)
    xp = jnp.pad(x, ((0, NP - N), (0, 0)))
    b1p = b1.reshape(1, HIDDEN)
    w2p = jnp.pad(W2, ((0, 0), (0, DA - C)))

    degs = _deg_pass(ep)

    t1, dinv = pl.pallas_call(
        _tc1_body,
        grid=(GRID,),
        in_specs=[
            pl.BlockSpec((BLK, F_IN), lambda i: (i, 0)),
            pl.BlockSpec((F_IN, HIDDEN), lambda i: (0, 0)),
            pl.BlockSpec((NW, BLK), lambda i: (0, i)),
        ],
        out_specs=[
            pl.BlockSpec((BLK, DA), lambda i: (i, 0)),
            pl.BlockSpec((BLK, 1), lambda i: (i, 0)),
        ],
        out_shape=[
            jax.ShapeDtypeStruct((NP, DA), jnp.float32),
            jax.ShapeDtypeStruct((NP, 1), jnp.float32),
        ],
    )(xp, W1, degs)

    agg1 = _agg_pass(t1, ep)

    t2 = pl.pallas_call(
        _tc2_body,
        grid=(GRID,),
        in_specs=[
            pl.BlockSpec((BLK, DA), lambda i: (i, 0)),
            pl.BlockSpec((BLK, DA), lambda i: (i, 0)),
            pl.BlockSpec((BLK, DA), lambda i: (i, 0)),
            pl.BlockSpec((BLK, 1), lambda i: (i, 0)),
            pl.BlockSpec((1, HIDDEN), lambda i: (0, 0)),
            pl.BlockSpec((HIDDEN, DA), lambda i: (0, 0)),
        ],
        out_specs=pl.BlockSpec((BLK, DA), lambda i: (i, 0)),
        out_shape=jax.ShapeDtypeStruct((NP, DA), jnp.float32),
    )(agg1[0], agg1[1], t1, dinv, b1p, w2p)

    agg2 = _agg_pass(t2, ep)

    outp = pl.pallas_call(
        _tc3_body,
        grid=(GRID,),
        in_specs=[
            pl.BlockSpec((BLK, DA), lambda i: (i, 0)),
            pl.BlockSpec((BLK, DA), lambda i: (i, 0)),
            pl.BlockSpec((BLK, DA), lambda i: (i, 0)),
            pl.BlockSpec((BLK, 1), lambda i: (i, 0)),
            pl.BlockSpec((1, C), lambda i: (0, 0)),
        ],
        out_specs=pl.BlockSpec((BLK, C), lambda i: (i, 0)),
        out_shape=jax.ShapeDtypeStruct((NP, C), jnp.float32),
    )(agg2[0], agg2[1], t2, dinv, b2.reshape(1, C))

    return outp[:N]


# speedup vs baseline: 3.9502x; 1.2751x over previous
import dataclasses
import functools

import jax
import jax.numpy as jnp
from jax import lax
from jax.experimental import pallas as pl
from jax.experimental.pallas import tpu as pltpu
from jax.experimental.pallas import tpu_sc as plsc

N = 10000
E = 320000
F_IN = 128
HIDDEN = 64
C = 40

NC = 2
NS = 16
LANES = 16
NW = NC * NS

CH = 128
CHN = 80
EPW = CH * CHN
E_PAD = EPW * NW
NP = 10240
ROWS_PER_SUB = NP // NS
DA = 64

BI = 8
TOT_CHUNKS = E_PAD // CH
NB0, NB1 = 11, 9
CPW0, CPW1 = NB0 * BI, NB1 * BI
assert NS * (CPW0 + CPW1) == TOT_CHUNKS

_MESH = plsc.VectorSubcoreMesh(core_axis_name="c", subcore_axis_name="s")

_CP = pltpu.CompilerParams()
if "needs_layout_passes" in pltpu.CompilerParams.__dataclass_fields__:
    _CP = dataclasses.replace(_CP, needs_layout_passes=False)
_CP_AGG = dataclasses.replace(pltpu.CompilerParams(),
                              use_tc_tiling_on_sc=False)


@functools.partial(
    pl.kernel,
    out_type=jax.ShapeDtypeStruct((NW, NP), jnp.float32),
    mesh=_MESH,
    scratch_types=[
        pltpu.VMEM((CHN, CH), jnp.int32),
        pltpu.VMEM((NP,), jnp.float32),
    ],
    compiler_params=_CP,
)
def _deg_pass(e_hbm, out_hbm, e_v, deg_v):
    c = lax.axis_index("c")
    s = lax.axis_index("s")
    wid = c * NS + s

    @pl.loop(0, NP // LANES)
    def _(i):
        deg_v[pl.ds(i * LANES, LANES)] = jnp.zeros((LANES,), jnp.float32)

    pltpu.sync_copy(e_hbm.at[1].at[pl.ds(wid * CHN, CHN)], e_v)

    @pl.loop(0, CHN)
    def _(j):
        @pl.loop(0, CH // LANES)
        def _(k):
            idx = e_v[j, pl.ds(k * LANES, LANES)]
            cnt, last = plsc.scan_count(idx)
            plsc.addupdate_scatter(deg_v, [idx], cnt.astype(jnp.float32),
                                   mask=last)

    pltpu.sync_copy(deg_v, out_hbm.at[wid])


W2A = 2 * DA


@functools.partial(
    pl.kernel,
    out_type=jax.ShapeDtypeStruct((NC, NP, W2A), jnp.float32),
    mesh=_MESH,
    scratch_types=[
        pltpu.VMEM((BI, CH), jnp.int32),
        pltpu.VMEM((BI, CH), jnp.int32),
        pltpu.VMEM((CH, DA), jnp.float32),
        pltpu.VMEM((CH, DA), jnp.float32),
        pltpu.VMEM_SHARED((NP, DA), jnp.float32),
        pltpu.VMEM_SHARED((NP, DA), jnp.float32),
        pltpu.SemaphoreType.DMA,
        pltpu.SemaphoreType.DMA,
        pltpu.SemaphoreType.DMA,
        pltpu.SemaphoreType.DMA,
    ],
    compiler_params=_CP_AGG,
)
def _agg_pass(t_hbm, e_hbm, out_hbm, src_v, dst_v, rows_a, rows_b,
              agg_sh, t_sh, gsem_a, gsem_b, ssem_a, ssem_b):
    c = lax.axis_index("c")
    s = lax.axis_index("s")

    @pl.loop(0, CH)
    def _(i):
        @pl.loop(0, DA // LANES)
        def _(k):
            rows_a[i, pl.ds(k * LANES, LANES)] = jnp.zeros((LANES,),
                                                           jnp.float32)

    @pl.loop(0, ROWS_PER_SUB // CH)
    def _(r):
        pltpu.sync_copy(rows_a,
                        agg_sh.at[pl.ds(s * ROWS_PER_SUB + r * CH, CH)])

    @pl.when(c == 1)
    def _():
        pltpu.sync_copy(t_hbm.at[pl.ds(s * ROWS_PER_SUB, ROWS_PER_SUB)],
                        t_sh.at[pl.ds(s * ROWS_PER_SUB, ROWS_PER_SUB)])

    plsc.subcore_barrier()

    def edge_loop(t_src, start_chunk, nblocks):
        @pl.loop(0, nblocks)
        def _(g):
            base = start_chunk + g * BI
            pltpu.sync_copy(e_hbm.at[0].at[pl.ds(base, BI)], src_v)
            pltpu.sync_copy(e_hbm.at[1].at[pl.ds(base, BI)], dst_v)
            bufs = ((rows_a, gsem_a, ssem_a), (rows_b, gsem_b, ssem_b))

            def gather(j, buf, gsem):
                pltpu.async_copy(t_src.at[src_v.at[j]], buf, gsem)

            def wait_gather(j, buf, gsem):
                pltpu.make_async_copy(t_src.at[src_v.at[j]], buf, gsem).wait()

            def scatter(j, buf, ssem):
                pltpu.async_copy(buf, agg_sh.at[dst_v.at[j]], ssem, add=True)

            def wait_scatter(j, buf, ssem):
                pltpu.make_async_copy(buf, agg_sh.at[dst_v.at[j]],
                                      ssem).wait()

            gather(0, *bufs[0][:2])
            gather(1, *bufs[1][:2])
            for j in range(BI):
                buf, gsem, ssem = bufs[j % 2]
                wait_gather(j, buf, gsem)
                scatter(j, buf, ssem)
                if j + 2 < BI:
                    wait_scatter(j, buf, ssem)
                    gather(j + 2, buf, gsem)
            wait_scatter(BI - 2, rows_a, ssem_a)
            wait_scatter(BI - 1, rows_b, ssem_b)

    @pl.when(c == 0)
    def _():
        edge_loop(t_hbm, s * CPW0, NB0)

    @pl.when(c == 1)
    def _():
        edge_loop(t_sh, NS * CPW0 + s * CPW1, NB1)

    plsc.subcore_barrier()

    @pl.loop(0, ROWS_PER_SUB // CH)
    def _(r):
        off = s * ROWS_PER_SUB + r * CH
        pltpu.sync_copy(agg_sh.at[pl.ds(off, CH)],
                        out_hbm.at[c].at[pl.ds(off, CH), pl.ds(0, DA)])


BLK = 1024
GRID = NP // BLK


def _tc1_body(x_ref, w1_ref, deg_ref, t1_ref, dinv_ref):
    deg = jnp.sum(deg_ref[...], axis=0)[:, None] + 1.0
    dinv = lax.rsqrt(deg)
    h = jnp.dot(x_ref[...], w1_ref[...], preferred_element_type=jnp.float32)
    t1_ref[...] = h * dinv
    dinv_ref[...] = dinv


def _tc2_body(aga_ref, agb_ref, t1_ref, dinv_ref, b1_ref, w2_ref, t2_ref):
    agg = aga_ref[:, :HIDDEN] + agb_ref[:, :HIDDEN] + t1_ref[...]
    out1 = agg * dinv_ref[...] + b1_ref[...]
    r = jnp.maximum(out1, 0.0)
    h2 = jnp.dot(r, w2_ref[...], preferred_element_type=jnp.float32)
    t2_ref[...] = h2 * dinv_ref[...]


def _tc3_body(aga_ref, agb_ref, t2_ref, dinv_ref, b2_ref, out_ref):
    z = (aga_ref[:, :C] + agb_ref[:, :C] + t2_ref[:, :C]) * dinv_ref[...]
    z40 = z + b2_ref[...]
    m = jnp.max(z40, axis=1, keepdims=True)
    lse = jnp.log(jnp.sum(jnp.exp(z40 - m), axis=1, keepdims=True))
    out_ref[...] = z40 - m - lse


def kernel(x, edge_index, W1, b1, W2, b2):
    pad = E_PAD - E
    fill = jnp.full((2, pad), N, jnp.int32)
    ep = jnp.concatenate([edge_index, fill], axis=1).reshape(
        2, TOT_CHUNKS, CH)
    xp = jnp.pad(x, ((0, NP - N), (0, 0)))
    b1p = b1.reshape(1, HIDDEN)
    w2p = jnp.pad(W2, ((0, 0), (0, DA - C)))

    degs = _deg_pass(ep)

    t1, dinv = pl.pallas_call(
        _tc1_body,
        grid=(GRID,),
        in_specs=[
            pl.BlockSpec((BLK, F_IN), lambda i: (i, 0)),
            pl.BlockSpec((F_IN, HIDDEN), lambda i: (0, 0)),
            pl.BlockSpec((NW, BLK), lambda i: (0, i)),
        ],
        out_specs=[
            pl.BlockSpec((BLK, DA), lambda i: (i, 0)),
            pl.BlockSpec((BLK, 1), lambda i: (i, 0)),
        ],
        out_shape=[
            jax.ShapeDtypeStruct((NP, DA), jnp.float32),
            jax.ShapeDtypeStruct((NP, 1), jnp.float32),
        ],
    )(xp, W1, degs)

    agg1 = _agg_pass(t1, ep)

    t2 = pl.pallas_call(
        _tc2_body,
        grid=(GRID,),
        in_specs=[
            pl.BlockSpec((BLK, W2A), lambda i: (i, 0)),
            pl.BlockSpec((BLK, W2A), lambda i: (i, 0)),
            pl.BlockSpec((BLK, DA), lambda i: (i, 0)),
            pl.BlockSpec((BLK, 1), lambda i: (i, 0)),
            pl.BlockSpec((1, HIDDEN), lambda i: (0, 0)),
            pl.BlockSpec((HIDDEN, DA), lambda i: (0, 0)),
        ],
        out_specs=pl.BlockSpec((BLK, DA), lambda i: (i, 0)),
        out_shape=jax.ShapeDtypeStruct((NP, DA), jnp.float32),
    )(agg1[0], agg1[1], t1, dinv, b1p, w2p)

    agg2 = _agg_pass(t2, ep)

    outp = pl.pallas_call(
        _tc3_body,
        grid=(GRID,),
        in_specs=[
            pl.BlockSpec((BLK, W2A), lambda i: (i, 0)),
            pl.BlockSpec((BLK, W2A), lambda i: (i, 0)),
            pl.BlockSpec((BLK, DA), lambda i: (i, 0)),
            pl.BlockSpec((BLK, 1), lambda i: (i, 0)),
            pl.BlockSpec((1, C), lambda i: (0, 0)),
        ],
        out_specs=pl.BlockSpec((BLK, C), lambda i: (i, 0)),
        out_shape=jax.ShapeDtypeStruct((NP, C), jnp.float32),
    )(agg2[0], agg2[1], t2, dinv, b2.reshape(1, C))

    return outp[:N]
